# baseline (speedup 1.0000x reference)
import functools

import jax
import jax.numpy as jnp
from jax import lax
from jax.experimental import pallas as pl
from jax.experimental.pallas import tpu as pltpu
from jax.experimental.pallas import tpu_sc as plsc

_NC = 2
_NS = 16
_NW = _NC * _NS
_BB = 16


def _make_repack(vocab, dim):
    n_full = vocab // 128
    rem = vocab % 128
    n_iters = n_full // _NW
    tail = n_full - n_iters * _NW
    mesh = plsc.VectorSubcoreMesh(core_axis_name="c", subcore_axis_name="s")

    @functools.partial(
        pl.kernel,
        mesh=mesh,
        out_type=jax.ShapeDtypeStruct((vocab * dim,), jnp.float32),
        scratch_types=[
            pltpu.VMEM((dim, 128), jnp.float32),
            pltpu.VMEM((dim, 128), jnp.float32),
            pltpu.VMEM((128 * dim,), jnp.float32),
            pltpu.VMEM((128 * dim,), jnp.float32),
            pltpu.SemaphoreType.DMA,
            pltpu.SemaphoreType.DMA,
            pltpu.SemaphoreType.DMA,
            pltpu.SemaphoreType.DMA,
        ],
        compiler_params=pltpu.CompilerParams(needs_layout_passes=False),
    )
    def k(wt_hbm, tail_hbm, out_hbm, in0, in1, ob0, ob1, gi0, gi1, go0, go1):
        wid = lax.axis_index("s") * _NC + lax.axis_index("c")
        iota64 = lax.broadcasted_iota(jnp.int32, (16,), 0) * dim

        def blk_of(g):
            return g * _NW + wid

        def in_desc(g, buf, sem):
            return (wt_hbm.at[:, pl.ds(blk_of(g) * 128, 128)], buf, sem)

        def out_desc(g, obuf, sem):
            return (obuf, out_hbm.at[pl.ds(blk_of(g) * 128 * dim, 128 * dim)], sem)

        def transpose(buf, obuf, lanes):
            def jbody(j, _):
                for l in range(lanes // 16):
                    v = buf[j, pl.ds(l * 16, 16)]
                    idx = iota64 + (l * 16 * dim + j)
                    plsc.store_scatter(obuf, [idx], v)
                return 0

            lax.fori_loop(0, dim, jbody, 0)

        n_total = n_iters + 1

        def valid(g):
            return blk_of(g) < n_full

        src0, dst0, sem0 = in_desc(0, in0, gi0)
        pltpu.async_copy(src0, dst0, gi0)

        def handle(g, ibuf, gi, obuf, go, ibuf_q, gi_q, go_q):
            src, dst, _ = in_desc(g, ibuf, gi)
            pltpu.make_async_copy(src, dst, gi).wait()

            @pl.when(valid(g + 1))
            def _():
                s, d, _ = in_desc(g + 1, ibuf_q, gi_q)
                pltpu.async_copy(s, d, gi_q)

            @pl.when(g >= 2)
            def _():
                s, d, _ = out_desc(g - 2, obuf, go)
                pltpu.make_async_copy(s, d, go).wait()

            transpose(ibuf, obuf, 128)

            s, d, _ = out_desc(g, obuf, go)
            pltpu.async_copy(s, d, go)

        def body(g, _):
            even = (g % 2) == 0

            @pl.when(valid(g))
            def _():
                @pl.when(even)
                def _():
                    handle(g, in0, gi0, ob0, go0, in1, gi1, go1)

                @pl.when(jnp.logical_not(even))
                def _():
                    handle(g, in1, gi1, ob1, go1, in0, gi0, go0)

            return 0

        lax.fori_loop(0, n_total, body, 0)

        last = jnp.where(wid < tail, n_iters, n_iters - 1)
        for g in (last - 1, last):
            @pl.when(g >= 0)
            def _():
                even = (g % 2) == 0

                @pl.when(even)
                def _():
                    s, d, _ = out_desc(g, ob0, go0)
                    pltpu.make_async_copy(s, d, go0).wait()

                @pl.when(jnp.logical_not(even))
                def _():
                    s, d, _ = out_desc(g, ob1, go1)
                    pltpu.make_async_copy(s, d, go1).wait()

        if rem:
            @pl.when(wid == _NW - 1)
            def _():
                pltpu.sync_copy(tail_hbm, in0)
                transpose(in0, ob0, rem)
                pltpu.sync_copy(
                    ob0.at[pl.ds(0, rem * dim)],
                    out_hbm.at[pl.ds(n_full * 128 * dim, rem * dim)],
                )

    return k


def _make_gather(batch, seq, dim, vocab):
    b_per_w = batch // _NW
    nblk = b_per_w // _BB
    assert nblk % 2 == 0
    mesh = plsc.VectorSubcoreMesh(core_axis_name="c", subcore_axis_name="s")

    @functools.partial(
        pl.kernel,
        mesh=mesh,
        out_type=jax.ShapeDtypeStruct((batch, seq, dim), jnp.float32),
        scratch_types=[
            pltpu.VMEM((b_per_w, seq), jnp.int32),
            pltpu.VMEM((_BB, seq, dim), jnp.float32),
            pltpu.VMEM((_BB, seq, dim), jnp.float32),
            pltpu.SemaphoreType.DMA,
            pltpu.SemaphoreType.DMA,
            pltpu.SemaphoreType.DMA,
            pltpu.SemaphoreType.DMA,
        ],
        compiler_params=pltpu.CompilerParams(use_tc_tiling_on_sc=False),
    )
    def k(idx_hbm, table_hbm, out_hbm, idx_v, rows0, rows1, g0, g1, o0, o1):
        wid = lax.axis_index("s") * _NC + lax.axis_index("c")
        base = wid * b_per_w
        pltpu.sync_copy(idx_hbm.at[wid], idx_v)

        def gather_descs(blk, rows, gsem):
            return [
                (table_hbm.at[idx_v.at[blk * _BB + t]], rows.at[t], gsem)
                for t in range(_BB)
            ]

        def out_slice(blk):
            return out_hbm.at[pl.ds(base + blk * _BB, _BB)]

        for src, dst, sem in gather_descs(0, rows0, g0):
            pltpu.async_copy(src, dst, sem)

        def handle(i, rows_p, gsem_p, osem_p, rows_q, gsem_q, osem_q):
            for src, dst, sem in gather_descs(i, rows_p, gsem_p):
                pltpu.make_async_copy(src, dst, sem).wait()
            pltpu.async_copy(rows_p, out_slice(i), osem_p)

            @pl.when(i + 1 < nblk)
            def _():
                @pl.when(i >= 1)
                def _():
                    pltpu.make_async_copy(rows_q, out_slice(i - 1), osem_q).wait()

                for src, dst, sem in gather_descs(i + 1, rows_q, gsem_q):
                    pltpu.async_copy(src, dst, sem)

        def body(i, _):
            even = (i % 2) == 0

            @pl.when(even)
            def _():
                handle(i, rows0, g0, o0, rows1, g1, o1)

            @pl.when(jnp.logical_not(even))
            def _():
                handle(i, rows1, g1, o1, rows0, g0, o0)

            return 0

        lax.fori_loop(0, nblk, body, 0)
        pltpu.make_async_copy(rows0, out_slice(nblk - 2), o0).wait()
        pltpu.make_async_copy(rows1, out_slice(nblk - 1), o1).wait()

    return k


@functools.partial(jax.jit, static_argnames=("batch", "seq", "dim", "vocab"))
def _impl(input_, weight, batch, seq, dim, vocab):
    n_full = vocab // 128
    rem = vocab % 128
    tail = jnp.zeros((dim, 128), jnp.float32)
    if rem:
        tail = tail.at[:, :rem].set(weight[n_full * 128:, :].T)
    w_lin = _make_repack(vocab, dim)(weight.T, tail)
    table = w_lin.reshape(vocab, dim)
    idx_3d = input_.reshape(_NW, batch // _NW, seq)
    return _make_gather(batch, seq, dim, vocab)(idx_3d, table)


def kernel(input_, weight):
    b, s = input_.shape
    vocab, dim = weight.shape
    return _impl(input_, weight, b, s, dim, vocab)


# speedup vs baseline: 1.2311x; 1.2311x over previous
import functools

import jax
import jax.numpy as jnp
from jax import lax
from jax.experimental import pallas as pl
from jax.experimental.pallas import tpu as pltpu
from jax.experimental.pallas import tpu_sc as plsc

_NC = 2
_NS = 16
_NW = _NC * _NS
_BB = 16


def _make_repack(vocab, dim):
    n_full = vocab // 128
    rem = vocab % 128
    n_iters = n_full // _NW
    tail = n_full - n_iters * _NW
    mesh = plsc.VectorSubcoreMesh(core_axis_name="c", subcore_axis_name="s")

    @functools.partial(
        pl.kernel,
        mesh=mesh,
        out_type=jax.ShapeDtypeStruct((vocab * dim,), jnp.float32),
        scratch_types=[
            pltpu.VMEM((dim, 128), jnp.float32),
            pltpu.VMEM((dim, 128), jnp.float32),
            pltpu.VMEM((128 * dim,), jnp.float32),
            pltpu.VMEM((128 * dim,), jnp.float32),
            pltpu.SemaphoreType.DMA,
            pltpu.SemaphoreType.DMA,
            pltpu.SemaphoreType.DMA,
            pltpu.SemaphoreType.DMA,
        ],
        compiler_params=pltpu.CompilerParams(needs_layout_passes=False),
    )
    def k(wt_hbm, tail_hbm, out_hbm, in0, in1, ob0, ob1, gi0, gi1, go0, go1):
        wid = lax.axis_index("s") * _NC + lax.axis_index("c")
        iota64 = lax.broadcasted_iota(jnp.int32, (16,), 0) * dim

        def blk_of(g):
            return g * _NW + wid

        def in_desc(g, buf, sem):
            return (wt_hbm.at[:, pl.ds(blk_of(g) * 128, 128)], buf, sem)

        def out_desc(g, obuf, sem):
            return (obuf, out_hbm.at[pl.ds(blk_of(g) * 128 * dim, 128 * dim)], sem)

        def transpose(buf, obuf, lanes):
            def jbody(j):
                for l in range(lanes // 16):
                    v = buf[j, pl.ds(l * 16, 16)]
                    idx = iota64 + (l * 16 * dim + j)
                    plsc.store_scatter(obuf, [idx], v)

            plsc.parallel_loop(0, dim, 1, unroll=4)(jbody)

        n_total = n_iters + 1

        def valid(g):
            return blk_of(g) < n_full

        src0, dst0, sem0 = in_desc(0, in0, gi0)
        pltpu.async_copy(src0, dst0, gi0)

        def handle(g, ibuf, gi, obuf, go, ibuf_q, gi_q, go_q):
            src, dst, _ = in_desc(g, ibuf, gi)
            pltpu.make_async_copy(src, dst, gi).wait()

            @pl.when(valid(g + 1))
            def _():
                s, d, _ = in_desc(g + 1, ibuf_q, gi_q)
                pltpu.async_copy(s, d, gi_q)

            @pl.when(g >= 2)
            def _():
                s, d, _ = out_desc(g - 2, obuf, go)
                pltpu.make_async_copy(s, d, go).wait()

            transpose(ibuf, obuf, 128)

            s, d, _ = out_desc(g, obuf, go)
            pltpu.async_copy(s, d, go)

        def body(g, _):
            even = (g % 2) == 0

            @pl.when(valid(g))
            def _():
                @pl.when(even)
                def _():
                    handle(g, in0, gi0, ob0, go0, in1, gi1, go1)

                @pl.when(jnp.logical_not(even))
                def _():
                    handle(g, in1, gi1, ob1, go1, in0, gi0, go0)

            return 0

        lax.fori_loop(0, n_total, body, 0)

        last = jnp.where(wid < tail, n_iters, n_iters - 1)
        for g in (last - 1, last):
            @pl.when(g >= 0)
            def _():
                even = (g % 2) == 0

                @pl.when(even)
                def _():
                    s, d, _ = out_desc(g, ob0, go0)
                    pltpu.make_async_copy(s, d, go0).wait()

                @pl.when(jnp.logical_not(even))
                def _():
                    s, d, _ = out_desc(g, ob1, go1)
                    pltpu.make_async_copy(s, d, go1).wait()

        if rem:
            @pl.when(wid == _NW - 1)
            def _():
                pltpu.sync_copy(tail_hbm, in0)
                transpose(in0, ob0, rem)
                pltpu.sync_copy(
                    ob0.at[pl.ds(0, rem * dim)],
                    out_hbm.at[pl.ds(n_full * 128 * dim, rem * dim)],
                )

    return k


def _make_gather(batch, seq, dim, vocab):
    b_per_w = batch // _NW
    nblk = b_per_w // _BB
    assert nblk % 2 == 0
    mesh = plsc.VectorSubcoreMesh(core_axis_name="c", subcore_axis_name="s")

    @functools.partial(
        pl.kernel,
        mesh=mesh,
        out_type=jax.ShapeDtypeStruct((batch, seq, dim), jnp.float32),
        scratch_types=[
            pltpu.VMEM((b_per_w, seq), jnp.int32),
            pltpu.VMEM((_BB, seq, dim), jnp.float32),
            pltpu.VMEM((_BB, seq, dim), jnp.float32),
            pltpu.SemaphoreType.DMA,
            pltpu.SemaphoreType.DMA,
            pltpu.SemaphoreType.DMA,
            pltpu.SemaphoreType.DMA,
        ],
        compiler_params=pltpu.CompilerParams(use_tc_tiling_on_sc=False),
    )
    def k(idx_hbm, table_hbm, out_hbm, idx_v, rows0, rows1, g0, g1, o0, o1):
        wid = lax.axis_index("s") * _NC + lax.axis_index("c")
        base = wid * b_per_w
        pltpu.sync_copy(idx_hbm.at[wid], idx_v)

        def gather_descs(blk, rows, gsem):
            return [
                (table_hbm.at[idx_v.at[blk * _BB + t]], rows.at[t], gsem)
                for t in range(_BB)
            ]

        def out_slice(blk):
            return out_hbm.at[pl.ds(base + blk * _BB, _BB)]

        for src, dst, sem in gather_descs(0, rows0, g0):
            pltpu.async_copy(src, dst, sem)

        def handle(i, rows_p, gsem_p, osem_p, rows_q, gsem_q, osem_q):
            for src, dst, sem in gather_descs(i, rows_p, gsem_p):
                pltpu.make_async_copy(src, dst, sem).wait()
            pltpu.async_copy(rows_p, out_slice(i), osem_p)

            @pl.when(i + 1 < nblk)
            def _():
                @pl.when(i >= 1)
                def _():
                    pltpu.make_async_copy(rows_q, out_slice(i - 1), osem_q).wait()

                for src, dst, sem in gather_descs(i + 1, rows_q, gsem_q):
                    pltpu.async_copy(src, dst, sem)

        def body(i, _):
            even = (i % 2) == 0

            @pl.when(even)
            def _():
                handle(i, rows0, g0, o0, rows1, g1, o1)

            @pl.when(jnp.logical_not(even))
            def _():
                handle(i, rows1, g1, o1, rows0, g0, o0)

            return 0

        lax.fori_loop(0, nblk, body, 0)
        pltpu.make_async_copy(rows0, out_slice(nblk - 2), o0).wait()
        pltpu.make_async_copy(rows1, out_slice(nblk - 1), o1).wait()

    return k


@functools.partial(jax.jit, static_argnames=("batch", "seq", "dim", "vocab"))
def _impl(input_, weight, batch, seq, dim, vocab):
    n_full = vocab // 128
    rem = vocab % 128
    tail = jnp.zeros((dim, 128), jnp.float32)
    if rem:
        tail = tail.at[:, :rem].set(weight[n_full * 128:, :].T)
    w_lin = _make_repack(vocab, dim)(weight.T, tail)
    table = w_lin.reshape(vocab, dim)
    idx_3d = input_.reshape(_NW, batch // _NW, seq)
    return _make_gather(batch, seq, dim, vocab)(idx_3d, table)


def kernel(input_, weight):
    b, s = input_.shape
    vocab, dim = weight.shape
    return _impl(input_, weight, b, s, dim, vocab)


# speedup vs baseline: 1.5501x; 1.2591x over previous
import functools

import jax
import jax.numpy as jnp
from jax import lax
from jax.experimental import pallas as pl
from jax.experimental.pallas import tpu as pltpu
from jax.experimental.pallas import tpu_sc as plsc

_NC = 2
_NS = 16
_NW = _NC * _NS
_GROUP = 128
_K = 4
_BLOCK = _GROUP * _K


@functools.partial(jax.jit, static_argnames=("n_total", "dim"))
def _gather(idx_flat, weight, n_total, dim):
    n_per_w = n_total // _NW
    n_groups = n_per_w // _GROUP
    nblk = n_groups // _K
    assert nblk % 2 == 0
    idx_3d = idx_flat.reshape(_NW, n_groups, _GROUP)

    mesh = plsc.VectorSubcoreMesh(core_axis_name="c", subcore_axis_name="s")

    @functools.partial(
        pl.kernel,
        mesh=mesh,
        out_type=jax.ShapeDtypeStruct((n_total, dim), jnp.float32),
        scratch_types=[
            pltpu.VMEM((n_groups, _GROUP), jnp.int32),
            pltpu.VMEM((_BLOCK, dim), jnp.float32),
            pltpu.VMEM((_BLOCK, dim), jnp.float32),
            pltpu.SemaphoreType.DMA,
            pltpu.SemaphoreType.DMA,
            pltpu.SemaphoreType.DMA,
            pltpu.SemaphoreType.DMA,
        ],
        compiler_params=pltpu.CompilerParams(use_tc_tiling_on_sc=False),
    )
    def k(idx_hbm, table_hbm, out_hbm, idx_v, rows0, rows1, g0, g1, o0, o1):
        wid = lax.axis_index("s") * _NC + lax.axis_index("c")
        base = wid * n_per_w
        pltpu.sync_copy(idx_hbm.at[wid], idx_v)

        def gather_descs(blk, rows, gsem):
            return [
                (table_hbm.at[idx_v.at[blk * _K + b]],
                 rows.at[pl.ds(b * _GROUP, _GROUP)],
                 gsem)
                for b in range(_K)
            ]

        def out_slice(blk):
            return out_hbm.at[pl.ds(base + blk * _BLOCK, _BLOCK)]

        for src, dst, sem in gather_descs(0, rows0, g0):
            pltpu.async_copy(src, dst, sem)

        def handle(i, rows_p, gsem_p, osem_p, rows_q, gsem_q, osem_q):
            for src, dst, sem in gather_descs(i, rows_p, gsem_p):
                pltpu.make_async_copy(src, dst, sem).wait()
            pltpu.async_copy(rows_p, out_slice(i), osem_p)

            @pl.when(i + 1 < nblk)
            def _():
                @pl.when(i >= 1)
                def _():
                    pltpu.make_async_copy(rows_q, out_slice(i - 1), osem_q).wait()

                for src, dst, sem in gather_descs(i + 1, rows_q, gsem_q):
                    pltpu.async_copy(src, dst, sem)

        def body(i, _):
            even = (i % 2) == 0

            @pl.when(even)
            def _():
                handle(i, rows0, g0, o0, rows1, g1, o1)

            @pl.when(jnp.logical_not(even))
            def _():
                handle(i, rows1, g1, o1, rows0, g0, o0)

            return 0

        lax.fori_loop(0, nblk, body, 0)
        pltpu.make_async_copy(rows0, out_slice(nblk - 2), o0).wait()
        pltpu.make_async_copy(rows1, out_slice(nblk - 1), o1).wait()

    return k(idx_3d, weight)


def kernel(input_, weight):
    b, s = input_.shape
    dim = weight.shape[1]
    n_total = b * s
    out_sm = _gather(input_.T.reshape(n_total), weight, n_total, dim)
    return out_sm.reshape(s, b, dim).transpose(1, 0, 2)
